# baseline (speedup 1.0000x reference)
import jax
import jax.numpy as jnp
from jax.experimental import pallas as pl


def _body(row_ref, col_ref, o_ref):
    r = row_ref[...]
    c = col_ref[...]
    d, h = r.shape
    w = c.shape[1]
    s = r[:, :, None] + c[:, None, :]
    o_ref[0] = s.reshape(d, h * w)


def kernel(x, mask, row_embed, col_embed):
    B = x.shape[0]
    h, w = x.shape[-2], x.shape[-1]
    d = row_embed.shape[-1]
    rowT = row_embed.T
    colT = col_embed.T
    out = pl.pallas_call(
        _body,
        grid=(B,),
        in_specs=[
            pl.BlockSpec((d, h), lambda b: (0, 0)),
            pl.BlockSpec((d, w), lambda b: (0, 0)),
        ],
        out_specs=pl.BlockSpec((1, d, h * w), lambda b: (b, 0, 0)),
        out_shape=jax.ShapeDtypeStruct((B, d, h * w), jnp.float32),
    )(rowT, colT)
    return out.reshape(B, d, h, w)


# speedup vs baseline: 1.7912x; 1.7912x over previous
import jax
import jax.numpy as jnp
from jax.experimental import pallas as pl
from jax.experimental.pallas import tpu as pltpu


def _body(row_ref, col_ref, o_ref, s_ref):
    d, h = row_ref.shape
    w = col_ref.shape[1]
    hw = h * w

    @pl.when(pl.program_id(0) == 0)
    def _():
        p_i = jax.lax.broadcasted_iota(jnp.int32, (h, hw), 1) // w
        p_j = jax.lax.broadcasted_iota(jnp.int32, (w, hw), 1) % w
        ii = jax.lax.broadcasted_iota(jnp.int32, (h, hw), 0)
        jj = jax.lax.broadcasted_iota(jnp.int32, (w, hw), 0)
        R = (p_i == ii).astype(jnp.float32)
        C = (p_j == jj).astype(jnp.float32)
        s_ref[...] = (
            jnp.dot(row_ref[...], R, preferred_element_type=jnp.float32)
            + jnp.dot(col_ref[...], C, preferred_element_type=jnp.float32)
        )

    o_ref[0] = s_ref[...]


def kernel(x, mask, row_embed, col_embed):
    B = x.shape[0]
    h, w = x.shape[-2], x.shape[-1]
    d = row_embed.shape[-1]
    rowT = row_embed.T
    colT = col_embed.T
    out = pl.pallas_call(
        _body,
        grid=(B,),
        in_specs=[
            pl.BlockSpec((d, h), lambda b: (0, 0)),
            pl.BlockSpec((d, w), lambda b: (0, 0)),
        ],
        out_specs=pl.BlockSpec((1, d, h * w), lambda b: (b, 0, 0)),
        out_shape=jax.ShapeDtypeStruct((B, d, h * w), jnp.float32),
        scratch_shapes=[pltpu.VMEM((d, h * w), jnp.float32)],
    )(rowT, colT)
    return out.reshape(B, d, h, w)


# speedup vs baseline: 1.8885x; 1.0543x over previous
import jax
import jax.numpy as jnp
from jax.experimental import pallas as pl
from jax.experimental.pallas import tpu as pltpu


def _body(row_ref, col_ref, o_ref, s_ref, sem):
    d, h = row_ref.shape
    w = col_ref.shape[1]
    hw = h * w
    B = o_ref.shape[0]

    p_i = jax.lax.broadcasted_iota(jnp.int32, (h, hw), 1) // w
    p_j = jax.lax.broadcasted_iota(jnp.int32, (w, hw), 1) % w
    ii = jax.lax.broadcasted_iota(jnp.int32, (h, hw), 0)
    jj = jax.lax.broadcasted_iota(jnp.int32, (w, hw), 0)
    R = (p_i == ii).astype(jnp.float32)
    C = (p_j == jj).astype(jnp.float32)
    s_ref[...] = (
        jnp.dot(row_ref[...], R, preferred_element_type=jnp.float32,
                precision=jax.lax.Precision.HIGHEST)
        + jnp.dot(col_ref[...], C, preferred_element_type=jnp.float32,
                  precision=jax.lax.Precision.HIGHEST)
    )

    copies = [pltpu.make_async_copy(s_ref, o_ref.at[b], sem) for b in range(B)]
    for c in copies:
        c.start()
    for c in copies:
        c.wait()


def kernel(x, mask, row_embed, col_embed):
    B = x.shape[0]
    h, w = x.shape[-2], x.shape[-1]
    d = row_embed.shape[-1]
    rowT = row_embed.T
    colT = col_embed.T
    out = pl.pallas_call(
        _body,
        in_specs=[
            pl.BlockSpec((d, h), lambda: (0, 0)),
            pl.BlockSpec((d, w), lambda: (0, 0)),
        ],
        out_specs=pl.BlockSpec(memory_space=pl.ANY),
        out_shape=jax.ShapeDtypeStruct((B, d, h * w), jnp.float32),
        scratch_shapes=[
            pltpu.VMEM((d, h * w), jnp.float32),
            pltpu.SemaphoreType.DMA,
        ],
    )(rowT, colT)
    return out.reshape(B, d, h, w)
